# baseline (speedup 1.0000x reference)
import functools

import jax
import jax.numpy as jnp
from jax import lax
from jax.experimental import pallas as pl
from jax.experimental.pallas import tpu as pltpu
from jax.experimental.pallas import tpu_sc as plsc

DAY_LEN = 288
D_MODEL = 64

NC = 2
NS = 16
L = 16
NW = NC * NS

DT = D_MODEL // 8
REP = 1
ROWW = DAY_LEN * REP


def _sc_lookup(n_b: int, n_t: int):
  b_per_w = n_b // NW
  bt_per_w = b_per_w // 128
  n_bt = n_b // 128
  mesh = plsc.VectorSubcoreMesh(
      core_axis_name="c", subcore_axis_name="s", num_cores=NC,
      num_subcores=NS)

  @functools.partial(
      pl.kernel,
      mesh=mesh,
      compiler_params=pltpu.CompilerParams(
          use_tc_tiling_on_sc=False, needs_layout_passes=False),
      out_type=jax.ShapeDtypeStruct((n_t, DT, n_bt, 8, 128), jnp.float32),
      scratch_types=[
          pltpu.VMEM((D_MODEL * ROWW,), jnp.float32),
          pltpu.VMEM((2, b_per_w), jnp.float32),
          pltpu.VMEM((2, DT, 2, 8, 128), jnp.float32),
          pltpu.SemaphoreType.DMA,
          pltpu.SemaphoreType.DMA,
          pltpu.SemaphoreType.DMA,
          pltpu.SemaphoreType.DMA,
      ],
  )
  def k(xt_hbm, table_hbm, out_hbm, table_v, x_v, stage_v, sx0, sx1, so0, so1):
    sem_x = (sx0, sx1)
    sem_o = (so0, so1)
    wid = lax.axis_index("s") * NC + lax.axis_index("c")
    b0 = wid * b_per_w
    bt0 = wid * bt_per_w

    pltpu.sync_copy(table_hbm, table_v)
    lane_sub = lax.rem(lax.iota(jnp.int32, L), jnp.int32(REP))

    def x_copy(t, xb):
      return pltpu.make_async_copy(
          xt_hbm.at[t, pl.ds(b0, b_per_w)], x_v.at[xb], sem_x[xb])

    def out_copy(t, h):
      return pltpu.make_async_copy(
          stage_v.at[h],
          out_hbm.at[t, :, pl.ds(bt0 + h * 2, 2), :, :], sem_o[h])

    def half(t, xb):
      x_copy(t, xb).wait()

      @pl.when(t + 1 < n_t)
      def _():
        x_copy(t + 1, 1 - xb).start()

      for h in range(2):
        @pl.when(t >= 1)
        def _(h=h):
          out_copy(t - 1, h).wait()

        @plsc.parallel_loop(0, 2, step=1)
        def _(bh, h=h):
          boff = h * 256 + bh * 128
          for gr in range(8):
            xv = x_v[xb, pl.ds(boff + gr * L, L)]
            iv = jnp.clip((xv * float(DAY_LEN)).astype(jnp.int32), 0,
                          DAY_LEN - 1)
            ivr = iv * REP + lane_sub
            G = 16
            prev = None
            for g0 in range(0, D_MODEL, G):
              cur = [(d, plsc.load_gather(
                  table_v.at[pl.ds(d * ROWW, ROWW)], [ivr]))
                     for d in range(g0, g0 + G)]
              if prev is not None:
                for d, v in prev:
                  stage_v[h, d // 8, bh, d % 8, pl.ds(gr * L, L)] = v
              prev = cur
            for d, v in prev:
              stage_v[h, d // 8, bh, d % 8, pl.ds(gr * L, L)] = v
        out_copy(t, h).start()

    x_copy(0, 0).start()

    def outer(t2, carry):
      half(t2 * 2, 0)
      half(t2 * 2 + 1, 1)
      return carry

    lax.fori_loop(0, n_t // 2, outer, 0)
    out_copy(n_t - 1, 0).wait()
    out_copy(n_t - 1, 1).wait()

  return k


def kernel(x_time_norm, day_emb):
  n_b, n_t = x_time_norm.shape
  xt = jnp.transpose(x_time_norm)
  table_r = jnp.repeat(jnp.transpose(day_emb), REP, axis=1).reshape(-1)
  out5 = _sc_lookup(n_b, n_t)(xt, table_r)
  out = jnp.transpose(out5, (2, 4, 0, 1, 3))
  return out.reshape(n_b, n_t, D_MODEL)


# speedup vs baseline: 1.0914x; 1.0914x over previous
import functools

import jax
import jax.numpy as jnp
from jax import lax
from jax.experimental import pallas as pl
from jax.experimental.pallas import tpu as pltpu
from jax.experimental.pallas import tpu_sc as plsc

DAY_LEN = 288
D_MODEL = 64

NC = 2
NS = 16
L = 16
NW = NC * NS

DT = D_MODEL // 8
REP = 4
ROWW = DAY_LEN * REP


def _sc_lookup(n_b: int, n_t: int):
  b_per_w = n_b // NW
  bt_per_w = b_per_w // 128
  n_bt = n_b // 128
  mesh = plsc.VectorSubcoreMesh(
      core_axis_name="c", subcore_axis_name="s", num_cores=NC,
      num_subcores=NS)

  @functools.partial(
      pl.kernel,
      mesh=mesh,
      compiler_params=pltpu.CompilerParams(
          use_tc_tiling_on_sc=False, needs_layout_passes=False),
      out_type=jax.ShapeDtypeStruct((n_t, DT, n_bt, 8, 128), jnp.float32),
      scratch_types=[
          pltpu.VMEM((D_MODEL * ROWW,), jnp.float32),
          pltpu.VMEM((2, b_per_w), jnp.float32),
          pltpu.VMEM((2, DT, 2, 8, 128), jnp.float32),
          pltpu.SemaphoreType.DMA,
          pltpu.SemaphoreType.DMA,
          pltpu.SemaphoreType.DMA,
          pltpu.SemaphoreType.DMA,
      ],
  )
  def k(xt_hbm, table_hbm, out_hbm, table_v, x_v, stage_v, sx0, sx1, so0, so1):
    sem_x = (sx0, sx1)
    sem_o = (so0, so1)
    wid = lax.axis_index("s") * NC + lax.axis_index("c")
    b0 = wid * b_per_w
    bt0 = wid * bt_per_w

    pltpu.sync_copy(table_hbm, table_v)
    lane_sub = lax.rem(lax.iota(jnp.int32, L), jnp.int32(REP))

    def x_copy(t, xb):
      return pltpu.make_async_copy(
          xt_hbm.at[t, pl.ds(b0, b_per_w)], x_v.at[xb], sem_x[xb])

    def out_copy(t, h):
      return pltpu.make_async_copy(
          stage_v.at[h],
          out_hbm.at[t, :, pl.ds(bt0 + h * 2, 2), :, :], sem_o[h])

    def half(t, xb):
      x_copy(t, xb).wait()

      @pl.when(t + 1 < n_t)
      def _():
        x_copy(t + 1, 1 - xb).start()

      for h in range(2):
        @pl.when(t >= 1)
        def _(h=h):
          out_copy(t - 1, h).wait()

        @plsc.parallel_loop(0, 2, step=1)
        def _(bh, h=h):
          boff = h * 256 + bh * 128
          for gr in range(8):
            xv = x_v[xb, pl.ds(boff + gr * L, L)]
            iv = jnp.clip((xv * float(DAY_LEN)).astype(jnp.int32), 0,
                          DAY_LEN - 1)
            ivr = iv * REP + lane_sub
            G = 8
            prev = None
            for g0 in range(0, D_MODEL, G):
              cur = [(d, plsc.load_gather(
                  table_v.at[pl.ds(d * ROWW, ROWW)], [ivr]))
                     for d in range(g0, g0 + G)]
              if prev is not None:
                for d, v in prev:
                  stage_v[h, d // 8, bh, d % 8, pl.ds(gr * L, L)] = v
              prev = cur
            for d, v in prev:
              stage_v[h, d // 8, bh, d % 8, pl.ds(gr * L, L)] = v
        out_copy(t, h).start()

    x_copy(0, 0).start()

    def outer(t2, carry):
      half(t2 * 2, 0)
      half(t2 * 2 + 1, 1)
      return carry

    lax.fori_loop(0, n_t // 2, outer, 0)
    out_copy(n_t - 1, 0).wait()
    out_copy(n_t - 1, 1).wait()

  return k


def kernel(x_time_norm, day_emb):
  n_b, n_t = x_time_norm.shape
  xt = jnp.transpose(x_time_norm)
  table_r = jnp.repeat(jnp.transpose(day_emb), REP, axis=1).reshape(-1)
  out5 = _sc_lookup(n_b, n_t)(xt, table_r)
  out = jnp.transpose(out5, (2, 4, 0, 1, 3))
  return out.reshape(n_b, n_t, D_MODEL)


# speedup vs baseline: 1.1689x; 1.0710x over previous
import functools

import jax
import jax.numpy as jnp
from jax import lax
from jax.experimental import pallas as pl
from jax.experimental.pallas import tpu as pltpu
from jax.experimental.pallas import tpu_sc as plsc

DAY_LEN = 288
D_MODEL = 64

NC = 2
NS = 16
L = 16
NW = NC * NS

DT = D_MODEL // 8
REP = 4
ROWW = DAY_LEN * REP


def _sc_lookup(n_b: int, n_t: int):
  b_per_w = n_b // NW
  bt_per_w = b_per_w // 128
  n_bt = n_b // 128
  mesh = plsc.VectorSubcoreMesh(
      core_axis_name="c", subcore_axis_name="s", num_cores=NC,
      num_subcores=NS)

  @functools.partial(
      pl.kernel,
      mesh=mesh,
      compiler_params=pltpu.CompilerParams(
          use_tc_tiling_on_sc=False, needs_layout_passes=False),
      out_type=jax.ShapeDtypeStruct((n_t, DT, n_bt, 8, 128), jnp.float32),
      scratch_types=[
          pltpu.VMEM((D_MODEL * ROWW,), jnp.float32),
          pltpu.VMEM((2, b_per_w), jnp.float32),
          pltpu.VMEM((2, DT, 2, 8, 128), jnp.float32),
          pltpu.SemaphoreType.DMA,
          pltpu.SemaphoreType.DMA,
          pltpu.SemaphoreType.DMA,
          pltpu.SemaphoreType.DMA,
      ],
  )
  def k(xt_hbm, table_hbm, out_hbm, table_v, x_v, stage_v, sx0, sx1, so0, so1):
    sem_x = (sx0, sx1)
    sem_o = (so0, so1)
    wid = lax.axis_index("s") * NC + lax.axis_index("c")
    b0 = wid * b_per_w
    bt0 = wid * bt_per_w

    pltpu.sync_copy(table_hbm, table_v)
    lane_sub = lax.rem(lax.iota(jnp.int32, L), jnp.int32(REP))

    def x_copy(t, xb):
      return pltpu.make_async_copy(
          xt_hbm.at[t, pl.ds(b0, b_per_w)], x_v.at[xb], sem_x[xb])

    def out_copy(t, h):
      return pltpu.make_async_copy(
          stage_v.at[h],
          out_hbm.at[t, :, pl.ds(bt0 + h * 2, 2), :, :], sem_o[h])

    def half(t, xb):
      x_copy(t, xb).wait()

      @pl.when(t + 1 < n_t)
      def _():
        x_copy(t + 1, 1 - xb).start()

      for h in range(2):
        @pl.when(t >= 1)
        def _(h=h):
          out_copy(t - 1, h).wait()

        @plsc.parallel_loop(0, 2, step=1)
        def _(bh, h=h):
          boff = h * 256 + bh * 128
          for gr in range(8):
            xv = x_v[xb, pl.ds(boff + gr * L, L)]
            iv = jnp.clip((xv * float(DAY_LEN)).astype(jnp.int32), 0,
                          DAY_LEN - 1)
            ivr = iv * REP + lane_sub
            G = 4
            prev = None
            for g0 in range(0, D_MODEL, G):
              cur = [(d, plsc.load_gather(
                  table_v.at[pl.ds(d * ROWW, ROWW)], [ivr]))
                     for d in range(g0, g0 + G)]
              if prev is not None:
                for d, v in prev:
                  stage_v[h, d // 8, bh, d % 8, pl.ds(gr * L, L)] = v
              prev = cur
            for d, v in prev:
              stage_v[h, d // 8, bh, d % 8, pl.ds(gr * L, L)] = v
        out_copy(t, h).start()

    x_copy(0, 0).start()

    def outer(t2, carry):
      half(t2 * 2, 0)
      half(t2 * 2 + 1, 1)
      return carry

    lax.fori_loop(0, n_t // 2, outer, 0)
    out_copy(n_t - 1, 0).wait()
    out_copy(n_t - 1, 1).wait()

  return k


def kernel(x_time_norm, day_emb):
  n_b, n_t = x_time_norm.shape
  xt = jnp.transpose(x_time_norm)
  table_r = jnp.repeat(jnp.transpose(day_emb), REP, axis=1).reshape(-1)
  out5 = _sc_lookup(n_b, n_t)(xt, table_r)
  out = jnp.transpose(out5, (2, 4, 0, 1, 3))
  return out.reshape(n_b, n_t, D_MODEL)
